# baseline (speedup 1.0000x reference)
import functools
import jax
import jax.numpy as jnp
from jax import lax
from jax.experimental import pallas as pl
from jax.experimental.pallas import tpu as pltpu
from jax.experimental.pallas import tpu_sc as plsc

NC = 2
NS = 16
NW = NC * NS
F = 128
B = 128


def _mm_body(x_ref, wb_ref, wa_ref, wc_ref, bb_ref, ba_ref, bc_ref,
             hidden_ref, dense_ref):
    xv = x_ref[...]
    hidden_ref[...] = jnp.dot(xv, wb_ref[...], preferred_element_type=jnp.float32)
    wd = (wc_ref[...] - wa_ref[...]) * (1.0 / 3.0)
    bias = bb_ref[...] + (bc_ref[...] - ba_ref[...]) * (1.0 / 3.0)
    dense_ref[...] = jnp.dot(jnp.abs(xv), wd, preferred_element_type=jnp.float32) + bias


def _combine_body(p0_ref, p1_ref, d_ref, out_ref):
    out_ref[...] = p0_ref[0] + p1_ref[0] + d_ref[...]


def _scale_rows(buf, wv, j):
    def group_body(g, carry2):
        wvec16 = wv[j, pl.ds(g * 16, 16)]
        for i in range(16):
            e = g * 16 + i
            wbc = jnp.full((16,), wvec16[i], jnp.float32)
            for cc in range(F // 16):
                sl = pl.ds(cc * 16, 16)
                buf[e, sl] = buf[e, sl] * wbc
        return carry2

    lax.fori_loop(0, B // 16, group_body, 0)


def _edge_body(hidden_hbm, col_hbm, row_hbm, w_hbm, zeros_hbm, out_hbm,
               colv, rowv, wv, buf, acc, gsem, nsteps):
    c = lax.axis_index("c")
    s = lax.axis_index("s")
    wid = c * NS + s
    stripe = acc.shape[0] // NS

    pltpu.sync_copy(zeros_hbm, acc.at[pl.ds(s * stripe, stripe)])

    pltpu.sync_copy(col_hbm.at[wid], colv)
    pltpu.sync_copy(row_hbm.at[wid], rowv)
    pltpu.sync_copy(w_hbm.at[wid], wv)

    plsc.subcore_barrier()

    def batch_body(j, carry):
        pltpu.async_copy(hidden_hbm.at[colv.at[j]], buf, gsem).wait()
        _scale_rows(buf, wv, j)
        pltpu.sync_copy(buf, acc.at[rowv.at[j]], add=True)
        return carry

    lax.fori_loop(0, nsteps, batch_body, 0)

    plsc.subcore_barrier()

    pltpu.sync_copy(acc.at[pl.ds(s * stripe, stripe)],
                    out_hbm.at[c, pl.ds(s * stripe, stripe)])


def kernel(x, edge_index, edge_weight, w_b, w_a, w_c, b_b, b_a, b_c):
    n, f_in = x.shape
    f_out = w_b.shape[1]
    e = edge_weight.shape[0]

    row = edge_index[0].astype(jnp.int32)
    col = edge_index[1].astype(jnp.int32)
    w = edge_weight.astype(jnp.float32)

    per_w = -(-e // (NW * 2 * B)) * 2 * B
    e_pad = per_w * NW
    if e_pad != e:
        pad = e_pad - e
        row = jnp.concatenate([row, jnp.zeros((pad,), jnp.int32)])
        col = jnp.concatenate([col, jnp.zeros((pad,), jnp.int32)])
        w = jnp.concatenate([w, jnp.zeros((pad,), jnp.float32)])
    nsteps = per_w // B
    col3 = col.reshape(NW, nsteps, B)
    row3 = row.reshape(NW, nsteps, B)
    w3 = w.reshape(NW, nsteps, B)

    bm = 1000
    grid = n // bm
    hidden, dense = pl.pallas_call(
        _mm_body,
        grid=(grid,),
        in_specs=[
            pl.BlockSpec((bm, f_in), lambda i: (i, 0)),
            pl.BlockSpec((f_in, f_out), lambda i: (0, 0)),
            pl.BlockSpec((f_in, f_out), lambda i: (0, 0)),
            pl.BlockSpec((f_in, f_out), lambda i: (0, 0)),
            pl.BlockSpec((1, f_out), lambda i: (0, 0)),
            pl.BlockSpec((1, f_out), lambda i: (0, 0)),
            pl.BlockSpec((1, f_out), lambda i: (0, 0)),
        ],
        out_specs=[
            pl.BlockSpec((bm, f_out), lambda i: (i, 0)),
            pl.BlockSpec((bm, f_out), lambda i: (i, 0)),
        ],
        out_shape=[
            jax.ShapeDtypeStruct((n, f_out), jnp.float32),
            jax.ShapeDtypeStruct((n, f_out), jnp.float32),
        ],
    )(x, w_b, w_a, w_c, b_b, b_a, b_c)

    stripe = -(-n // (NS * 8)) * 8
    n_pad = stripe * NS
    zeros = jnp.zeros((stripe, f_out), jnp.float32)
    mesh = plsc.VectorSubcoreMesh(core_axis_name="c", subcore_axis_name="s",
                                  num_cores=NC, num_subcores=NS)
    partials = pl.kernel(
        functools.partial(_edge_body, nsteps=nsteps),
        out_type=jax.ShapeDtypeStruct((NC, n_pad, f_out), jnp.float32),
        mesh=mesh,
        scratch_types=[
            pltpu.VMEM((nsteps, B), jnp.int32),
            pltpu.VMEM((nsteps, B), jnp.int32),
            pltpu.VMEM((nsteps, B), jnp.float32),
            pltpu.VMEM((B, f_out), jnp.float32),
            pltpu.VMEM_SHARED((n_pad, f_out), jnp.float32),
            pltpu.SemaphoreType.DMA,
        ],
    )(hidden, col3, row3, w3, zeros)

    out = pl.pallas_call(
        _combine_body,
        grid=(grid,),
        in_specs=[
            pl.BlockSpec((1, bm, f_out), lambda i: (0, i, 0)),
            pl.BlockSpec((1, bm, f_out), lambda i: (1, i, 0)),
            pl.BlockSpec((bm, f_out), lambda i: (i, 0)),
        ],
        out_specs=pl.BlockSpec((bm, f_out), lambda i: (i, 0)),
        out_shape=jax.ShapeDtypeStruct((n, f_out), jnp.float32),
    )(partials, partials, dense)
    return out


# speedup vs baseline: 2.2001x; 2.2001x over previous
import functools
import jax
import jax.numpy as jnp
from jax import lax
from jax.experimental import pallas as pl
from jax.experimental.pallas import tpu as pltpu
from jax.experimental.pallas import tpu_sc as plsc

NC = 2
NS = 16
NW = NC * NS
F = 128
B = 128


def _mm_body(x_ref, wb_ref, wa_ref, wc_ref, bb_ref, ba_ref, bc_ref,
             hidden_ref, dense_ref):
    xv = x_ref[...]
    hidden_ref[...] = jnp.dot(xv, wb_ref[...], preferred_element_type=jnp.float32)
    wd = (wc_ref[...] - wa_ref[...]) * (1.0 / 3.0)
    bias = bb_ref[...] + (bc_ref[...] - ba_ref[...]) * (1.0 / 3.0)
    dense_ref[...] = jnp.dot(jnp.abs(xv), wd, preferred_element_type=jnp.float32) + bias


def _combine_body(p0_ref, p1_ref, d_ref, out_ref):
    out_ref[...] = p0_ref[0] + p1_ref[0] + d_ref[...]


def _scale_rows(buf, wv, j):
    def group_body(g, carry2):
        wvec16 = wv[j, pl.ds(g * 16, 16)]
        for i in range(16):
            e = g * 16 + i
            wbc = jnp.full((16,), wvec16[i], jnp.float32)
            for cc in range(F // 16):
                sl = pl.ds(cc * 16, 16)
                buf[e, sl] = buf[e, sl] * wbc
        return carry2

    lax.fori_loop(0, B // 16, group_body, 0)


def _edge_body(hidden_hbm, col_hbm, row_hbm, w_hbm, zeros_hbm, out_hbm,
               colv, rowv, wv, buf, acc, gsem, nsteps):
    c = lax.axis_index("c")
    s = lax.axis_index("s")
    wid = c * NS + s
    stripe = acc.shape[0] // NS

    pltpu.sync_copy(zeros_hbm, acc.at[pl.ds(s * stripe, stripe)])

    pltpu.sync_copy(col_hbm.at[wid], colv)
    pltpu.sync_copy(row_hbm.at[wid], rowv)
    pltpu.sync_copy(w_hbm.at[wid], wv)

    plsc.subcore_barrier()

    def batch_body(j, carry):
        pltpu.async_copy(hidden_hbm.at[colv.at[j]], buf, gsem).wait()
        _scale_rows(buf, wv, j)
        pltpu.sync_copy(buf, acc.at[rowv.at[j]], add=True)
        return carry

    lax.fori_loop(0, nsteps, batch_body, 0)

    plsc.subcore_barrier()

    pltpu.sync_copy(acc.at[pl.ds(s * stripe, stripe)],
                    out_hbm.at[c, pl.ds(s * stripe, stripe)])


def kernel(x, edge_index, edge_weight, w_b, w_a, w_c, b_b, b_a, b_c):
    n, f_in = x.shape
    f_out = w_b.shape[1]
    e = edge_weight.shape[0]

    row = edge_index[0].astype(jnp.int32)
    col = edge_index[1].astype(jnp.int32)
    w = edge_weight.astype(jnp.float32)

    stripe = -(-n // (NS * 8)) * 8
    n_pad = stripe * NS
    per_w = -(-e // (NW * B)) * B
    e_pad = per_w * NW
    if e_pad != e:
        pad = e_pad - e
        spare = max(n_pad - n, 1)
        pad_rows = n + (jnp.arange(pad, dtype=jnp.int32) % spare)
        pad_cols = jnp.arange(pad, dtype=jnp.int32) % n
        row = jnp.concatenate([row, pad_rows])
        col = jnp.concatenate([col, pad_cols])
        w = jnp.concatenate([w, jnp.zeros((pad,), jnp.float32)])
    nsteps = per_w // B
    col3 = col.reshape(NW, nsteps, B)
    row3 = row.reshape(NW, nsteps, B)
    w3 = w.reshape(NW, nsteps, B)

    bm = 1000
    grid = n // bm
    hidden, dense = pl.pallas_call(
        _mm_body,
        grid=(grid,),
        in_specs=[
            pl.BlockSpec((bm, f_in), lambda i: (i, 0)),
            pl.BlockSpec((f_in, f_out), lambda i: (0, 0)),
            pl.BlockSpec((f_in, f_out), lambda i: (0, 0)),
            pl.BlockSpec((f_in, f_out), lambda i: (0, 0)),
            pl.BlockSpec((1, f_out), lambda i: (0, 0)),
            pl.BlockSpec((1, f_out), lambda i: (0, 0)),
            pl.BlockSpec((1, f_out), lambda i: (0, 0)),
        ],
        out_specs=[
            pl.BlockSpec((bm, f_out), lambda i: (i, 0)),
            pl.BlockSpec((bm, f_out), lambda i: (i, 0)),
        ],
        out_shape=[
            jax.ShapeDtypeStruct((n, f_out), jnp.float32),
            jax.ShapeDtypeStruct((n, f_out), jnp.float32),
        ],
    )(x, w_b, w_a, w_c, b_b, b_a, b_c)

    stripe = -(-n // (NS * 8)) * 8
    n_pad = stripe * NS
    zeros = jnp.zeros((stripe, f_out), jnp.float32)
    mesh = plsc.VectorSubcoreMesh(core_axis_name="c", subcore_axis_name="s",
                                  num_cores=NC, num_subcores=NS)
    partials = pl.kernel(
        functools.partial(_edge_body, nsteps=nsteps),
        out_type=jax.ShapeDtypeStruct((NC, n_pad, f_out), jnp.float32),
        mesh=mesh,
        scratch_types=[
            pltpu.VMEM((nsteps, B), jnp.int32),
            pltpu.VMEM((nsteps, B), jnp.int32),
            pltpu.VMEM((nsteps, B), jnp.float32),
            pltpu.VMEM((B, f_out), jnp.float32),
            pltpu.VMEM_SHARED((n_pad, f_out), jnp.float32),
            pltpu.SemaphoreType.DMA,
        ],
    )(hidden, col3, row3, w3, zeros)

    out = pl.pallas_call(
        _combine_body,
        grid=(grid,),
        in_specs=[
            pl.BlockSpec((1, bm, f_out), lambda i: (0, i, 0)),
            pl.BlockSpec((1, bm, f_out), lambda i: (1, i, 0)),
            pl.BlockSpec((bm, f_out), lambda i: (i, 0)),
        ],
        out_specs=pl.BlockSpec((bm, f_out), lambda i: (i, 0)),
        out_shape=jax.ShapeDtypeStruct((n, f_out), jnp.float32),
    )(partials, partials, dense)
    return out
